# baseline (speedup 1.0000x reference)
import functools

import jax
import jax.numpy as jnp
from jax import lax
from jax.experimental import pallas as pl
from jax.experimental.pallas import tpu as pltpu
from jax.experimental.pallas import tpu_sc as plsc

N = 10000
F = 128
H = 64
L = 40
LP = 48
NNZ = 160000
E = 640000
ALPHA = 0.1
ITERS = 10

NC, NS = 2, 16
NW = NC * NS
RT = N // NS
SB = 512
KB = SB // 128
FPAD = 163840
EPAD = 655360


def _wbcast(w16, e):
    return w16.at[jnp.full((16,), e, jnp.int32)].get(mode="promise_in_bounds")


def _make_sc_spmm(M, D, T):
    per_w = M // NW
    rpw = per_w // 128
    nsb = per_w // SB
    tpt = T // NS
    mesh = plsc.VectorSubcoreMesh(core_axis_name="c", subcore_axis_name="s")

    @functools.partial(
        pl.kernel,
        mesh=mesh,
        compiler_params=pltpu.CompilerParams(use_tc_tiling_on_sc=False),
        out_type=pltpu.HBM((NC, N, D), jnp.float32),
        scratch_types=[
            pltpu.VMEM((3, KB, 128), jnp.int32),
            pltpu.VMEM((3, KB, 128), jnp.int32),
            pltpu.VMEM((3, KB, 128), jnp.float32),
            pltpu.VMEM((2, KB, 128, D), jnp.float32),
            pltpu.VMEM((RT // 5, D), jnp.float32),
            pltpu.VMEM_SHARED((N, D), jnp.float32),
            pltpu.VMEM_SHARED((T, D), jnp.float32),
            pltpu.SemaphoreType.DMA((3,)),
            pltpu.SemaphoreType.DMA((2,)),
            pltpu.SemaphoreType.DMA((2,)),
        ],
    )
    def k(src_hbm, dst_hbm, w_hbm, table_hbm, q_out,
          src_v, dst_v, w_v, rows_v, a_v, acc_sh, tab_sh, sem_i, sem_g, sem_s):
        c = lax.axis_index("c")
        s = lax.axis_index("s")
        wid = s * NC + c
        RC = RT // 5

        def zrow(i, carry):
            for j in range(D // 16):
                a_v[i, pl.ds(j * 16, 16)] = jnp.zeros((16,), jnp.float32)
            return carry
        lax.fori_loop(0, RC, zrow, 0)
        for t in range(5):
            pltpu.sync_copy(a_v, acc_sh.at[pl.ds(s * RT + t * RC, RC)])
        pltpu.sync_copy(table_hbm.at[pl.ds(s * tpt, tpt)], tab_sh.at[pl.ds(s * tpt, tpt)])
        plsc.subcore_barrier()

        base = wid * rpw

        def stage_issue(sb, slot):
            row0 = base + sb * KB
            pltpu.async_copy(src_hbm.at[pl.ds(row0, KB)], src_v.at[slot], sem_i.at[slot])
            pltpu.async_copy(dst_hbm.at[pl.ds(row0, KB)], dst_v.at[slot], sem_i.at[slot])
            pltpu.async_copy(w_hbm.at[pl.ds(row0, KB)], w_v.at[slot], sem_i.at[slot])

        def stage_wait(sb, slot):
            row0 = base + sb * KB
            pltpu.make_async_copy(src_hbm.at[pl.ds(row0, KB)], src_v.at[slot], sem_i.at[slot]).wait()
            pltpu.make_async_copy(dst_hbm.at[pl.ds(row0, KB)], dst_v.at[slot], sem_i.at[slot]).wait()
            pltpu.make_async_copy(w_hbm.at[pl.ds(row0, KB)], w_v.at[slot], sem_i.at[slot]).wait()

        def gathers_issue(rslot, islot):
            for k2 in range(KB):
                pltpu.async_copy(tab_sh.at[src_v.at[islot, k2]], rows_v.at[rslot, k2], sem_g.at[rslot])

        def gathers_wait(rslot, islot):
            for k2 in range(KB):
                pltpu.make_async_copy(tab_sh.at[src_v.at[islot, k2]], rows_v.at[rslot, k2], sem_g.at[rslot]).wait()

        def scatters_issue(rslot, islot):
            for k2 in range(KB):
                pltpu.async_copy(rows_v.at[rslot, k2], acc_sh.at[dst_v.at[islot, k2]], sem_s.at[rslot], add=True)

        def scatters_wait(rslot, islot):
            for k2 in range(KB):
                pltpu.make_async_copy(rows_v.at[rslot, k2], acc_sh.at[dst_v.at[islot, k2]], sem_s.at[rslot]).wait()

        stage_issue(0, 0)
        stage_issue(1, 1)
        stage_wait(0, 0)
        gathers_issue(0, 0)

        def sb_step(sb, carry):
            pb = sb % 2
            m = sb % 3
            m1 = (sb + 1) % 3
            m2 = (sb + 2) % 3
            gathers_wait(pb, m)

            @pl.when(sb >= 1)
            def _drain_prev():
                scatters_wait(1 - pb, m2)

            @pl.when(sb + 1 < nsb)
            def _prefetch():
                stage_wait(sb + 1, m1)
                gathers_issue(1 - pb, m1)

            def panel(k2, carry2):
                def grp(g, carry3):
                    w16 = w_v[m, k2, pl.ds(g * 16, 16)]
                    for e in range(16):
                        wb = _wbcast(w16, e)
                        for j in range(D // 16):
                            sl = pl.ds(j * 16, 16)
                            rows_v[pb, k2, g * 16 + e, sl] = rows_v[pb, k2, g * 16 + e, sl] * wb
                    return carry3
                return lax.fori_loop(0, 8, grp, carry2)
            lax.fori_loop(0, KB, panel, 0)

            scatters_issue(pb, m)

            @pl.when(sb + 2 < nsb)
            def _stage_next():
                stage_issue(sb + 2, m2)
            return carry
        lax.fori_loop(0, nsb, sb_step, 0)
        scatters_wait((nsb - 1) % 2, (nsb - 1) % 3)
        plsc.subcore_barrier()

        for t in range(5):
            rs = pl.ds(s * RT + t * RC, RC)
            pltpu.sync_copy(acc_sh.at[rs], a_v)
            pltpu.sync_copy(a_v, q_out.at[c, rs])

    return k


_fspmm = _make_sc_spmm(FPAD, H, F)
_pspmm = _make_sc_spmm(EPAD, LP, N)


_R = 2000


def _dense_body(p_ref, b1_ref, w2_ref, b2_ref, out_ref):
    l1 = jnp.maximum(p_ref[0] + p_ref[1] + b1_ref[...], 0.0)
    out_ref[...] = (
        jnp.dot(l1, w2_ref[...], preferred_element_type=jnp.float32) + b2_ref[...]
    )


def _dense(p, b1, w2p, b2p):
    return pl.pallas_call(
        _dense_body,
        grid=(N // _R,),
        in_specs=[
            pl.BlockSpec((NC, _R, H), lambda i: (0, i, 0)),
            pl.BlockSpec((1, H), lambda i: (0, 0)),
            pl.BlockSpec((H, LP), lambda i: (0, 0)),
            pl.BlockSpec((1, LP), lambda i: (0, 0)),
        ],
        out_specs=pl.BlockSpec((_R, LP), lambda i: (i, 0)),
        out_shape=jax.ShapeDtypeStruct((N, LP), jnp.float32),
    )(p, b1, w2p, b2p)


def _comb_body(q_ref, l2_ref, out_ref):
    out_ref[...] = (1.0 - ALPHA) * (q_ref[0] + q_ref[1]) + ALPHA * l2_ref[...]


def _combine(q, l2):
    return pl.pallas_call(
        _comb_body,
        grid=(N // _R,),
        in_specs=[
            pl.BlockSpec((NC, _R, LP), lambda i: (0, i, 0)),
            pl.BlockSpec((_R, LP), lambda i: (i, 0)),
        ],
        out_specs=pl.BlockSpec((_R, LP), lambda i: (i, 0)),
        out_shape=jax.ShapeDtypeStruct((N, LP), jnp.float32),
    )(q, l2)


def _lsm_body(q_ref, l2_ref, out_ref):
    x = (1.0 - ALPHA) * (q_ref[0] + q_ref[1]) + ALPHA * l2_ref[...]
    x = x[:, :L]
    m = jnp.max(x, axis=1, keepdims=True)
    e = jnp.exp(x - m)
    lse = jnp.log(jnp.sum(e, axis=1, keepdims=True)) + m
    out_ref[...] = x - lse


def _lsm(q, l2):
    return pl.pallas_call(
        _lsm_body,
        grid=(N // _R,),
        in_specs=[
            pl.BlockSpec((NC, _R, LP), lambda i: (0, i, 0)),
            pl.BlockSpec((_R, LP), lambda i: (i, 0)),
        ],
        out_specs=pl.BlockSpec((_R, L), lambda i: (i, 0)),
        out_shape=jax.ShapeDtypeStruct((N, L), jnp.float32),
    )(q, l2)


def _pad2d(x, m, dtype):
    return jnp.concatenate([x, jnp.zeros((m - x.shape[0],), dtype)]).reshape(-1, 128)


def kernel(feature_indices, feature_values, edge_indices, edge_weights, W1, b1, W2, b2):
    f_src = _pad2d(feature_indices[1], FPAD, jnp.int32)
    f_dst = _pad2d(feature_indices[0], FPAD, jnp.int32)
    f_w = _pad2d(feature_values, FPAD, jnp.float32)
    e_src = _pad2d(edge_indices[1], EPAD, jnp.int32)
    e_dst = _pad2d(edge_indices[0], EPAD, jnp.int32)
    e_w = _pad2d(edge_weights, EPAD, jnp.float32)

    p = _fspmm(f_src, f_dst, f_w, W1)

    w2p = jnp.pad(W2, ((0, 0), (0, LP - L)))
    b2p = jnp.pad(b2, (0, LP - L)).reshape(1, LP)
    l2 = _dense(p, b1.reshape(1, H), w2p, b2p)

    loc = l2
    for i in range(ITERS):
        q = _pspmm(e_src, e_dst, e_w, loc)
        if i + 1 < ITERS:
            loc = _combine(q, l2)
    return _lsm(q, l2)


# speedup vs baseline: 1.0700x; 1.0700x over previous
import functools

import jax
import jax.numpy as jnp
from jax import lax
from jax.experimental import pallas as pl
from jax.experimental.pallas import tpu as pltpu
from jax.experimental.pallas import tpu_sc as plsc

N = 10000
F = 128
H = 64
L = 40
LP = 48
NNZ = 160000
E = 640000
ALPHA = 0.1
ITERS = 10

NC, NS = 2, 16
NW = NC * NS
RT = N // NS
SB = 512
KB = SB // 128
FPAD = 163840
EPAD = 655360


def _wbcast(w16, e):
    return w16.at[jnp.full((16,), e, jnp.int32)].get(mode="promise_in_bounds")


def _make_sc_spmm(M, D, T):
    per_w = M // NW
    rpw = per_w // 128
    nsb = per_w // SB
    tpt = T // NS
    mesh = plsc.VectorSubcoreMesh(core_axis_name="c", subcore_axis_name="s")

    @functools.partial(
        pl.kernel,
        mesh=mesh,
        compiler_params=pltpu.CompilerParams(use_tc_tiling_on_sc=False),
        out_type=pltpu.HBM((NC, N, D), jnp.float32),
        scratch_types=[
            pltpu.VMEM((3, KB, 128), jnp.int32),
            pltpu.VMEM((3, KB, 128), jnp.int32),
            pltpu.VMEM((3, KB, 128), jnp.float32),
            pltpu.VMEM((2, KB, 128, D), jnp.float32),
            pltpu.VMEM((RT // 5, D), jnp.float32),
            pltpu.VMEM_SHARED((N, D), jnp.float32),
            pltpu.VMEM_SHARED((T, D), jnp.float32),
            pltpu.SemaphoreType.DMA((3,)),
            pltpu.SemaphoreType.DMA((2,)),
            pltpu.SemaphoreType.DMA((2,)),
        ],
    )
    def k(src_hbm, dst_hbm, w_hbm, table_hbm, q_out,
          src_v, dst_v, w_v, rows_v, a_v, acc_sh, tab_sh, sem_i, sem_g, sem_s):
        c = lax.axis_index("c")
        s = lax.axis_index("s")
        wid = s * NC + c
        RC = RT // 5

        def zrow(i, carry):
            for j in range(D // 16):
                a_v[i, pl.ds(j * 16, 16)] = jnp.zeros((16,), jnp.float32)
            return carry
        lax.fori_loop(0, RC, zrow, 0)
        for t in range(5):
            pltpu.sync_copy(a_v, acc_sh.at[pl.ds(s * RT + t * RC, RC)])
        pltpu.sync_copy(table_hbm.at[pl.ds(s * tpt, tpt)], tab_sh.at[pl.ds(s * tpt, tpt)])
        plsc.subcore_barrier()

        base = wid * rpw

        def stage_issue(sb, slot):
            row0 = base + sb * KB
            pltpu.async_copy(src_hbm.at[pl.ds(row0, KB)], src_v.at[slot], sem_i.at[slot])
            pltpu.async_copy(dst_hbm.at[pl.ds(row0, KB)], dst_v.at[slot], sem_i.at[slot])
            pltpu.async_copy(w_hbm.at[pl.ds(row0, KB)], w_v.at[slot], sem_i.at[slot])

        def stage_wait(sb, slot):
            row0 = base + sb * KB
            pltpu.make_async_copy(src_hbm.at[pl.ds(row0, KB)], src_v.at[slot], sem_i.at[slot]).wait()
            pltpu.make_async_copy(dst_hbm.at[pl.ds(row0, KB)], dst_v.at[slot], sem_i.at[slot]).wait()
            pltpu.make_async_copy(w_hbm.at[pl.ds(row0, KB)], w_v.at[slot], sem_i.at[slot]).wait()

        def gathers_issue(rslot, islot):
            for k2 in range(KB):
                pltpu.async_copy(tab_sh.at[src_v.at[islot, k2]], rows_v.at[rslot, k2], sem_g.at[rslot])

        def gathers_wait(rslot, islot):
            for k2 in range(KB):
                pltpu.make_async_copy(tab_sh.at[src_v.at[islot, k2]], rows_v.at[rslot, k2], sem_g.at[rslot]).wait()

        def scatters_issue(rslot, islot):
            for k2 in range(KB):
                pltpu.async_copy(rows_v.at[rslot, k2], acc_sh.at[dst_v.at[islot, k2]], sem_s.at[rslot], add=True)

        def scatters_wait(rslot, islot):
            for k2 in range(KB):
                pltpu.make_async_copy(rows_v.at[rslot, k2], acc_sh.at[dst_v.at[islot, k2]], sem_s.at[rslot]).wait()

        stage_issue(0, 0)
        stage_wait(0, 0)
        gathers_issue(0, 0)
        stage_issue(1, 1)

        def sb_step(sb, carry):
            pb = sb % 2
            gathers_wait(pb, pb)

            @pl.when(sb + 1 < nsb)
            def _prefetch():
                stage_wait(sb + 1, 1 - pb)
                gathers_issue(1 - pb, 1 - pb)

            def panel(k2, carry2):
                def grp(g, carry3):
                    w16 = w_v[pb, k2, pl.ds(g * 16, 16)]
                    for e in range(16):
                        wb = _wbcast(w16, e)
                        for j in range(D // 16):
                            sl = pl.ds(j * 16, 16)
                            rows_v[pb, k2, g * 16 + e, sl] = rows_v[pb, k2, g * 16 + e, sl] * wb
                    return carry3
                return lax.fori_loop(0, 8, grp, carry2)
            lax.fori_loop(0, KB, panel, 0)

            scatters_issue(pb, pb)
            scatters_wait(pb, pb)

            @pl.when(sb + 2 < nsb)
            def _stage_next():
                stage_issue(sb + 2, pb)
            return carry
        lax.fori_loop(0, nsb, sb_step, 0)
        plsc.subcore_barrier()

        for t in range(5):
            rs = pl.ds(s * RT + t * RC, RC)
            pltpu.sync_copy(acc_sh.at[rs], a_v)
            pltpu.sync_copy(a_v, q_out.at[c, rs])

    return k


_fspmm = _make_sc_spmm(FPAD, H, F)
_pspmm = _make_sc_spmm(EPAD, LP, N)


_R = 2000


def _dense_body(p_ref, b1_ref, w2_ref, b2_ref, out_ref):
    l1 = jnp.maximum(p_ref[0] + p_ref[1] + b1_ref[...], 0.0)
    out_ref[...] = (
        jnp.dot(l1, w2_ref[...], preferred_element_type=jnp.float32) + b2_ref[...]
    )


def _dense(p, b1, w2p, b2p):
    return pl.pallas_call(
        _dense_body,
        grid=(N // _R,),
        in_specs=[
            pl.BlockSpec((NC, _R, H), lambda i: (0, i, 0)),
            pl.BlockSpec((1, H), lambda i: (0, 0)),
            pl.BlockSpec((H, LP), lambda i: (0, 0)),
            pl.BlockSpec((1, LP), lambda i: (0, 0)),
        ],
        out_specs=pl.BlockSpec((_R, LP), lambda i: (i, 0)),
        out_shape=jax.ShapeDtypeStruct((N, LP), jnp.float32),
    )(p, b1, w2p, b2p)


def _comb_body(q_ref, l2_ref, out_ref):
    out_ref[...] = (1.0 - ALPHA) * (q_ref[0] + q_ref[1]) + ALPHA * l2_ref[...]


def _combine(q, l2):
    return pl.pallas_call(
        _comb_body,
        grid=(N // _R,),
        in_specs=[
            pl.BlockSpec((NC, _R, LP), lambda i: (0, i, 0)),
            pl.BlockSpec((_R, LP), lambda i: (i, 0)),
        ],
        out_specs=pl.BlockSpec((_R, LP), lambda i: (i, 0)),
        out_shape=jax.ShapeDtypeStruct((N, LP), jnp.float32),
    )(q, l2)


def _lsm_body(q_ref, l2_ref, out_ref):
    x = (1.0 - ALPHA) * (q_ref[0] + q_ref[1]) + ALPHA * l2_ref[...]
    x = x[:, :L]
    m = jnp.max(x, axis=1, keepdims=True)
    e = jnp.exp(x - m)
    lse = jnp.log(jnp.sum(e, axis=1, keepdims=True)) + m
    out_ref[...] = x - lse


def _lsm(q, l2):
    return pl.pallas_call(
        _lsm_body,
        grid=(N // _R,),
        in_specs=[
            pl.BlockSpec((NC, _R, LP), lambda i: (0, i, 0)),
            pl.BlockSpec((_R, LP), lambda i: (i, 0)),
        ],
        out_specs=pl.BlockSpec((_R, L), lambda i: (i, 0)),
        out_shape=jax.ShapeDtypeStruct((N, L), jnp.float32),
    )(q, l2)


def _pad2d(x, m, dtype):
    return jnp.concatenate([x, jnp.zeros((m - x.shape[0],), dtype)]).reshape(-1, 128)


def kernel(feature_indices, feature_values, edge_indices, edge_weights, W1, b1, W2, b2):
    f_src = _pad2d(feature_indices[1], FPAD, jnp.int32)
    f_dst = _pad2d(feature_indices[0], FPAD, jnp.int32)
    f_w = _pad2d(feature_values, FPAD, jnp.float32)
    e_src = _pad2d(edge_indices[1], EPAD, jnp.int32)
    e_dst = _pad2d(edge_indices[0], EPAD, jnp.int32)
    e_w = _pad2d(edge_weights, EPAD, jnp.float32)

    p = _fspmm(f_src, f_dst, f_w, W1)

    w2p = jnp.pad(W2, ((0, 0), (0, LP - L)))
    b2p = jnp.pad(b2, (0, LP - L)).reshape(1, LP)
    l2 = _dense(p, b1.reshape(1, H), w2p, b2p)

    loc = l2
    for i in range(ITERS):
        q = _pspmm(e_src, e_dst, e_w, loc)
        if i + 1 < ITERS:
            loc = _combine(q, l2)
    return _lsm(q, l2)


# speedup vs baseline: 1.0740x; 1.0037x over previous
import functools

import jax
import jax.numpy as jnp
from jax import lax
from jax.experimental import pallas as pl
from jax.experimental.pallas import tpu as pltpu
from jax.experimental.pallas import tpu_sc as plsc

N = 10000
F = 128
H = 64
L = 40
LP = 48
NNZ = 160000
E = 640000
ALPHA = 0.1
ITERS = 10

NC, NS = 2, 16
NW = NC * NS
RT = N // NS
SB = 512
KB = SB // 128
FPAD = 163840
EPAD = 655360


def _wbcast(w16, e):
    return w16.at[jnp.full((16,), e, jnp.int32)].get(mode="promise_in_bounds")


def _make_sc_spmm(M, D, T, combine=False):
    per_w = M // NW
    rpw = per_w // 128
    nsb = per_w // SB
    tpt = T // NS
    mesh = plsc.VectorSubcoreMesh(core_axis_name="c", subcore_axis_name="s")

    @functools.partial(
        pl.kernel,
        mesh=mesh,
        compiler_params=pltpu.CompilerParams(use_tc_tiling_on_sc=False),
        out_type=pltpu.HBM((NC, N, D), jnp.float32),
        scratch_types=[
            pltpu.VMEM((3, KB, 128), jnp.int32),
            pltpu.VMEM((3, KB, 128), jnp.int32),
            pltpu.VMEM((3, KB, 128), jnp.float32),
            pltpu.VMEM((2, KB, 128, D), jnp.float32),
            pltpu.VMEM((RT // 5, D), jnp.float32),
            pltpu.VMEM_SHARED((N, D), jnp.float32),
            pltpu.VMEM_SHARED((T, D), jnp.float32),
            pltpu.SemaphoreType.DMA((3,)),
            pltpu.SemaphoreType.DMA((2,)),
            pltpu.SemaphoreType.DMA((2,)),
        ] + ([pltpu.VMEM((RT // 5, D), jnp.float32)] if combine else []),
    )
    def k(src_hbm, dst_hbm, w_hbm, table_hbm, *rest):
        if combine:
            (q_hbm, q_out, src_v, dst_v, w_v, rows_v, a_v, acc_sh, tab_sh,
             sem_i, sem_g, sem_s, b_v) = rest
        else:
            (q_out, src_v, dst_v, w_v, rows_v, a_v, acc_sh, tab_sh,
             sem_i, sem_g, sem_s) = rest
        c = lax.axis_index("c")
        s = lax.axis_index("s")
        wid = s * NC + c
        RC = RT // 5

        def zrow(i, carry):
            for j in range(D // 16):
                a_v[i, pl.ds(j * 16, 16)] = jnp.zeros((16,), jnp.float32)
            return carry
        lax.fori_loop(0, RC, zrow, 0)
        for t in range(5):
            pltpu.sync_copy(a_v, acc_sh.at[pl.ds(s * RT + t * RC, RC)])
        if combine:
            for t in range(5):
                rs = pl.ds(s * RT + t * RC, RC)
                pltpu.sync_copy(q_hbm.at[0, rs], a_v)
                pltpu.sync_copy(q_hbm.at[1, rs], b_v)

                def c1(i, carry):
                    for j in range(D // 16):
                        sl = pl.ds(j * 16, 16)
                        a_v[i, sl] = (1.0 - ALPHA) * (a_v[i, sl] + b_v[i, sl])
                    return carry
                lax.fori_loop(0, RC, c1, 0)
                pltpu.sync_copy(table_hbm.at[rs], b_v)

                def c2(i, carry):
                    for j in range(D // 16):
                        sl = pl.ds(j * 16, 16)
                        a_v[i, sl] = a_v[i, sl] + ALPHA * b_v[i, sl]
                    return carry
                lax.fori_loop(0, RC, c2, 0)
                pltpu.sync_copy(a_v, tab_sh.at[rs])
        else:
            pltpu.sync_copy(table_hbm.at[pl.ds(s * tpt, tpt)], tab_sh.at[pl.ds(s * tpt, tpt)])
        plsc.subcore_barrier()

        base = wid * rpw

        def stage_issue(sb, slot):
            row0 = base + sb * KB
            pltpu.async_copy(src_hbm.at[pl.ds(row0, KB)], src_v.at[slot], sem_i.at[slot])
            pltpu.async_copy(dst_hbm.at[pl.ds(row0, KB)], dst_v.at[slot], sem_i.at[slot])
            pltpu.async_copy(w_hbm.at[pl.ds(row0, KB)], w_v.at[slot], sem_i.at[slot])

        def stage_wait(sb, slot):
            row0 = base + sb * KB
            pltpu.make_async_copy(src_hbm.at[pl.ds(row0, KB)], src_v.at[slot], sem_i.at[slot]).wait()
            pltpu.make_async_copy(dst_hbm.at[pl.ds(row0, KB)], dst_v.at[slot], sem_i.at[slot]).wait()
            pltpu.make_async_copy(w_hbm.at[pl.ds(row0, KB)], w_v.at[slot], sem_i.at[slot]).wait()

        def gathers_issue(rslot, islot):
            for k2 in range(KB):
                pltpu.async_copy(tab_sh.at[src_v.at[islot, k2]], rows_v.at[rslot, k2], sem_g.at[rslot])

        def gathers_wait(rslot, islot):
            for k2 in range(KB):
                pltpu.make_async_copy(tab_sh.at[src_v.at[islot, k2]], rows_v.at[rslot, k2], sem_g.at[rslot]).wait()

        def scatters_issue(rslot, islot):
            for k2 in range(KB):
                pltpu.async_copy(rows_v.at[rslot, k2], acc_sh.at[dst_v.at[islot, k2]], sem_s.at[rslot], add=True)

        def scatters_wait(rslot, islot):
            for k2 in range(KB):
                pltpu.make_async_copy(rows_v.at[rslot, k2], acc_sh.at[dst_v.at[islot, k2]], sem_s.at[rslot]).wait()

        stage_issue(0, 0)
        stage_wait(0, 0)
        gathers_issue(0, 0)
        stage_issue(1, 1)

        def sb_step(sb, carry):
            pb = sb % 2
            gathers_wait(pb, pb)

            @pl.when(sb + 1 < nsb)
            def _prefetch():
                stage_wait(sb + 1, 1 - pb)
                gathers_issue(1 - pb, 1 - pb)

            def panel(k2, carry2):
                def grp(g, carry3):
                    w16 = w_v[pb, k2, pl.ds(g * 16, 16)]
                    for e in range(16):
                        wb = _wbcast(w16, e)
                        for j in range(D // 16):
                            sl = pl.ds(j * 16, 16)
                            rows_v[pb, k2, g * 16 + e, sl] = rows_v[pb, k2, g * 16 + e, sl] * wb
                    return carry3
                return lax.fori_loop(0, 8, grp, carry2)
            lax.fori_loop(0, KB, panel, 0)

            scatters_issue(pb, pb)
            scatters_wait(pb, pb)

            @pl.when(sb + 2 < nsb)
            def _stage_next():
                stage_issue(sb + 2, pb)
            return carry
        lax.fori_loop(0, nsb, sb_step, 0)
        plsc.subcore_barrier()

        for t in range(5):
            rs = pl.ds(s * RT + t * RC, RC)
            pltpu.sync_copy(acc_sh.at[rs], a_v)
            pltpu.sync_copy(a_v, q_out.at[c, rs])

    return k


_fspmm = _make_sc_spmm(FPAD, H, F)
_pspmm = _make_sc_spmm(EPAD, LP, N, combine=True)


_R = 2000


def _dense_body(p_ref, b1_ref, w2_ref, b2_ref, out_ref):
    l1 = jnp.maximum(p_ref[0] + p_ref[1] + b1_ref[...], 0.0)
    out_ref[...] = (
        jnp.dot(l1, w2_ref[...], preferred_element_type=jnp.float32) + b2_ref[...]
    )


def _dense(p, b1, w2p, b2p):
    return pl.pallas_call(
        _dense_body,
        grid=(N // _R,),
        in_specs=[
            pl.BlockSpec((NC, _R, H), lambda i: (0, i, 0)),
            pl.BlockSpec((1, H), lambda i: (0, 0)),
            pl.BlockSpec((H, LP), lambda i: (0, 0)),
            pl.BlockSpec((1, LP), lambda i: (0, 0)),
        ],
        out_specs=pl.BlockSpec((_R, LP), lambda i: (i, 0)),
        out_shape=jax.ShapeDtypeStruct((N, LP), jnp.float32),
    )(p, b1, w2p, b2p)


def _comb_body(q_ref, l2_ref, out_ref):
    out_ref[...] = (1.0 - ALPHA) * (q_ref[0] + q_ref[1]) + ALPHA * l2_ref[...]


def _combine(q, l2):
    return pl.pallas_call(
        _comb_body,
        grid=(N // _R,),
        in_specs=[
            pl.BlockSpec((NC, _R, LP), lambda i: (0, i, 0)),
            pl.BlockSpec((_R, LP), lambda i: (i, 0)),
        ],
        out_specs=pl.BlockSpec((_R, LP), lambda i: (i, 0)),
        out_shape=jax.ShapeDtypeStruct((N, LP), jnp.float32),
    )(q, l2)


def _lsm_body(q_ref, l2_ref, out_ref):
    x = (1.0 - ALPHA) * (q_ref[0] + q_ref[1]) + ALPHA * l2_ref[...]
    x = x[:, :L]
    m = jnp.max(x, axis=1, keepdims=True)
    e = jnp.exp(x - m)
    lse = jnp.log(jnp.sum(e, axis=1, keepdims=True)) + m
    out_ref[...] = x - lse


def _lsm(q, l2):
    return pl.pallas_call(
        _lsm_body,
        grid=(N // _R,),
        in_specs=[
            pl.BlockSpec((NC, _R, LP), lambda i: (0, i, 0)),
            pl.BlockSpec((_R, LP), lambda i: (i, 0)),
        ],
        out_specs=pl.BlockSpec((_R, L), lambda i: (i, 0)),
        out_shape=jax.ShapeDtypeStruct((N, L), jnp.float32),
    )(q, l2)


def _pad2d(x, m, dtype):
    return jnp.concatenate([x, jnp.zeros((m - x.shape[0],), dtype)]).reshape(-1, 128)


def kernel(feature_indices, feature_values, edge_indices, edge_weights, W1, b1, W2, b2):
    f_src = _pad2d(feature_indices[1], FPAD, jnp.int32)
    f_dst = _pad2d(feature_indices[0], FPAD, jnp.int32)
    f_w = _pad2d(feature_values, FPAD, jnp.float32)
    e_src = _pad2d(edge_indices[1], EPAD, jnp.int32)
    e_dst = _pad2d(edge_indices[0], EPAD, jnp.int32)
    e_w = _pad2d(edge_weights, EPAD, jnp.float32)

    p = _fspmm(f_src, f_dst, f_w, W1)

    w2p = jnp.pad(W2, ((0, 0), (0, LP - L)))
    b2p = jnp.pad(b2, (0, LP - L)).reshape(1, LP)
    l2 = _dense(p, b1.reshape(1, H), w2p, b2p)

    q = jnp.broadcast_to(0.5 * l2, (NC, N, LP))
    for _ in range(ITERS):
        q = _pspmm(e_src, e_dst, e_w, l2, q)
    return _lsm(q, l2)


# speedup vs baseline: 1.0797x; 1.0053x over previous
import functools

import jax
import jax.numpy as jnp
from jax import lax
from jax.experimental import pallas as pl
from jax.experimental.pallas import tpu as pltpu
from jax.experimental.pallas import tpu_sc as plsc

N = 10000
F = 128
H = 64
L = 40
LP = 48
NNZ = 160000
E = 640000
ALPHA = 0.1
ITERS = 10

NC, NS = 2, 16
NW = NC * NS
RT = N // NS
SB = 512
KB = SB // 128
FPAD = 163840
EPAD = 655360


def _wbcast(w16, e):
    return w16.at[jnp.full((16,), e, jnp.int32)].get(mode="promise_in_bounds")


def _make_sc_spmm(M, D, T, combine=False):
    per_w = M // NW
    rpw = per_w // 128
    nsb = per_w // SB
    tpt = T // NS
    mesh = plsc.VectorSubcoreMesh(core_axis_name="c", subcore_axis_name="s")

    @functools.partial(
        pl.kernel,
        mesh=mesh,
        compiler_params=pltpu.CompilerParams(use_tc_tiling_on_sc=False),
        out_type=pltpu.HBM((NC, N, D), jnp.float32),
        scratch_types=[
            pltpu.VMEM((3, KB, 128), jnp.int32),
            pltpu.VMEM((3, KB, 128), jnp.int32),
            pltpu.VMEM((3, KB, 128), jnp.float32),
            pltpu.VMEM((2, KB, 128, D), jnp.float32),
            pltpu.VMEM((RT // 5, D), jnp.float32),
            pltpu.VMEM_SHARED((N, D), jnp.float32),
            pltpu.VMEM_SHARED((T, D), jnp.float32),
            pltpu.SemaphoreType.DMA((3,)),
            pltpu.SemaphoreType.DMA((2,)),
            pltpu.SemaphoreType.DMA((2,)),
            pltpu.SemaphoreType.DMA((2,)),
            pltpu.SemaphoreType.DMA,
        ] + ([pltpu.VMEM((2, 3, RT // 25, D), jnp.float32)] if combine else []),
    )
    def k(src_hbm, dst_hbm, w_hbm, table_hbm, *rest):
        if combine:
            (q_hbm, q_out, src_v, dst_v, w_v, rows_v, a_v, acc_sh, tab_sh,
             sem_i, sem_g, sem_s, sem_p, sem_q, cb_v) = rest
        else:
            (q_out, src_v, dst_v, w_v, rows_v, a_v, acc_sh, tab_sh,
             sem_i, sem_g, sem_s, sem_p, sem_q) = rest
        c = lax.axis_index("c")
        s = lax.axis_index("s")
        wid = s * NC + c
        RC = RT // 5

        def zrow(i, carry):
            for j in range(D // 16):
                a_v[i, pl.ds(j * 16, 16)] = jnp.zeros((16,), jnp.float32)
            return carry
        lax.fori_loop(0, RC, zrow, 0)
        zh = [pltpu.async_copy(a_v, acc_sh.at[pl.ds(s * RT + t * RC, RC)], sem_q)
              for t in range(5)]
        for h in zh:
            h.wait()
        if combine:
            NCH = 25
            RC2 = RT // NCH

            def crs(t):
                return pl.ds(s * RT + t * RC2, RC2)

            def cload(t, sl):
                pltpu.async_copy(q_hbm.at[0, crs(t)], cb_v.at[sl, 0], sem_p.at[sl])
                pltpu.async_copy(q_hbm.at[1, crs(t)], cb_v.at[sl, 1], sem_p.at[sl])
                pltpu.async_copy(table_hbm.at[crs(t)], cb_v.at[sl, 2], sem_p.at[sl])

            def cload_wait(t, sl):
                pltpu.make_async_copy(q_hbm.at[0, crs(t)], cb_v.at[sl, 0], sem_p.at[sl]).wait()
                pltpu.make_async_copy(q_hbm.at[1, crs(t)], cb_v.at[sl, 1], sem_p.at[sl]).wait()
                pltpu.make_async_copy(table_hbm.at[crs(t)], cb_v.at[sl, 2], sem_p.at[sl]).wait()

            cload(0, 0)

            def cstep(t, carry):
                sl = t % 2

                @pl.when(t >= 1)
                def _drain_store():
                    pltpu.make_async_copy(cb_v.at[1 - sl, 0], tab_sh.at[crs(t - 1)],
                                          sem_q).wait()

                @pl.when(t + 1 < NCH)
                def _next_load():
                    cload(t + 1, 1 - sl)
                cload_wait(t, sl)

                def crow(i, carry2):
                    for j in range(D // 16):
                        csl = pl.ds(j * 16, 16)
                        cb_v[sl, 0, i, csl] = ((1.0 - ALPHA)
                                               * (cb_v[sl, 0, i, csl] + cb_v[sl, 1, i, csl])
                                               + ALPHA * cb_v[sl, 2, i, csl])
                    return carry2
                lax.fori_loop(0, RC2, crow, 0)
                pltpu.async_copy(cb_v.at[sl, 0], tab_sh.at[crs(t)], sem_q)
                return carry
            lax.fori_loop(0, NCH, cstep, 0)
            pltpu.make_async_copy(cb_v.at[(NCH - 1) % 2, 0], tab_sh.at[crs(NCH - 1)],
                                  sem_q).wait()
        else:
            pltpu.sync_copy(table_hbm.at[pl.ds(s * tpt, tpt)], tab_sh.at[pl.ds(s * tpt, tpt)])
        plsc.subcore_barrier()

        base = wid * rpw

        def stage_issue(sb, slot):
            row0 = base + sb * KB
            pltpu.async_copy(src_hbm.at[pl.ds(row0, KB)], src_v.at[slot], sem_i.at[slot])
            pltpu.async_copy(dst_hbm.at[pl.ds(row0, KB)], dst_v.at[slot], sem_i.at[slot])
            pltpu.async_copy(w_hbm.at[pl.ds(row0, KB)], w_v.at[slot], sem_i.at[slot])

        def stage_wait(sb, slot):
            row0 = base + sb * KB
            pltpu.make_async_copy(src_hbm.at[pl.ds(row0, KB)], src_v.at[slot], sem_i.at[slot]).wait()
            pltpu.make_async_copy(dst_hbm.at[pl.ds(row0, KB)], dst_v.at[slot], sem_i.at[slot]).wait()
            pltpu.make_async_copy(w_hbm.at[pl.ds(row0, KB)], w_v.at[slot], sem_i.at[slot]).wait()

        def gathers_issue(rslot, islot):
            for k2 in range(KB):
                pltpu.async_copy(tab_sh.at[src_v.at[islot, k2]], rows_v.at[rslot, k2], sem_g.at[rslot])

        def gathers_wait(rslot, islot):
            for k2 in range(KB):
                pltpu.make_async_copy(tab_sh.at[src_v.at[islot, k2]], rows_v.at[rslot, k2], sem_g.at[rslot]).wait()

        def scatters_issue(rslot, islot):
            for k2 in range(KB):
                pltpu.async_copy(rows_v.at[rslot, k2], acc_sh.at[dst_v.at[islot, k2]], sem_s.at[rslot], add=True)

        def scatters_wait(rslot, islot):
            for k2 in range(KB):
                pltpu.make_async_copy(rows_v.at[rslot, k2], acc_sh.at[dst_v.at[islot, k2]], sem_s.at[rslot]).wait()

        stage_issue(0, 0)
        stage_wait(0, 0)
        gathers_issue(0, 0)
        stage_issue(1, 1)

        def sb_step(sb, carry):
            pb = sb % 2
            gathers_wait(pb, pb)

            @pl.when(sb + 1 < nsb)
            def _prefetch():
                stage_wait(sb + 1, 1 - pb)
                gathers_issue(1 - pb, 1 - pb)

            def panel(k2, carry2):
                def grp(g, carry3):
                    w16 = w_v[pb, k2, pl.ds(g * 16, 16)]
                    for e in range(16):
                        wb = _wbcast(w16, e)
                        for j in range(D // 16):
                            sl = pl.ds(j * 16, 16)
                            rows_v[pb, k2, g * 16 + e, sl] = rows_v[pb, k2, g * 16 + e, sl] * wb
                    return carry3
                return lax.fori_loop(0, 8, grp, carry2)
            lax.fori_loop(0, KB, panel, 0)

            scatters_issue(pb, pb)
            scatters_wait(pb, pb)

            @pl.when(sb + 2 < nsb)
            def _stage_next():
                stage_issue(sb + 2, pb)
            return carry
        lax.fori_loop(0, nsb, sb_step, 0)
        plsc.subcore_barrier()

        if combine:
            NCH = 25
            RC2 = RT // NCH

            def rrs(t):
                return pl.ds(s * RT + t * RC2, RC2)

            def rread(t, sl):
                pltpu.async_copy(acc_sh.at[rrs(t)], cb_v.at[sl, 0], sem_p.at[sl])

            def rread_wait(t, sl):
                pltpu.make_async_copy(acc_sh.at[rrs(t)], cb_v.at[sl, 0], sem_p.at[sl]).wait()

            rread(0, 0)

            def rstep(t, carry):
                sl = t % 2

                @pl.when(t >= 1)
                def _drain_store():
                    pltpu.make_async_copy(cb_v.at[1 - sl, 0], q_out.at[c, rrs(t - 1)],
                                          sem_q).wait()

                @pl.when(t + 1 < NCH)
                def _next_read():
                    rread(t + 1, 1 - sl)
                rread_wait(t, sl)
                pltpu.async_copy(cb_v.at[sl, 0], q_out.at[c, rrs(t)], sem_q)
                return carry
            lax.fori_loop(0, NCH, rstep, 0)
            pltpu.make_async_copy(cb_v.at[(NCH - 1) % 2, 0], q_out.at[c, rrs(NCH - 1)],
                                  sem_q).wait()
        else:
            for t in range(5):
                rs = pl.ds(s * RT + t * RC, RC)
                pltpu.sync_copy(acc_sh.at[rs], a_v)
                pltpu.sync_copy(a_v, q_out.at[c, rs])

    return k


_fspmm = _make_sc_spmm(FPAD, H, F)
_pspmm = _make_sc_spmm(EPAD, LP, N, combine=True)


_R = 2000


def _dense_body(p_ref, b1_ref, w2_ref, b2_ref, out_ref):
    l1 = jnp.maximum(p_ref[0] + p_ref[1] + b1_ref[...], 0.0)
    out_ref[...] = (
        jnp.dot(l1, w2_ref[...], preferred_element_type=jnp.float32) + b2_ref[...]
    )


def _dense(p, b1, w2p, b2p):
    return pl.pallas_call(
        _dense_body,
        grid=(N // _R,),
        in_specs=[
            pl.BlockSpec((NC, _R, H), lambda i: (0, i, 0)),
            pl.BlockSpec((1, H), lambda i: (0, 0)),
            pl.BlockSpec((H, LP), lambda i: (0, 0)),
            pl.BlockSpec((1, LP), lambda i: (0, 0)),
        ],
        out_specs=pl.BlockSpec((_R, LP), lambda i: (i, 0)),
        out_shape=jax.ShapeDtypeStruct((N, LP), jnp.float32),
    )(p, b1, w2p, b2p)


def _comb_body(q_ref, l2_ref, out_ref):
    out_ref[...] = (1.0 - ALPHA) * (q_ref[0] + q_ref[1]) + ALPHA * l2_ref[...]


def _combine(q, l2):
    return pl.pallas_call(
        _comb_body,
        grid=(N // _R,),
        in_specs=[
            pl.BlockSpec((NC, _R, LP), lambda i: (0, i, 0)),
            pl.BlockSpec((_R, LP), lambda i: (i, 0)),
        ],
        out_specs=pl.BlockSpec((_R, LP), lambda i: (i, 0)),
        out_shape=jax.ShapeDtypeStruct((N, LP), jnp.float32),
    )(q, l2)


def _lsm_body(q_ref, l2_ref, out_ref):
    x = (1.0 - ALPHA) * (q_ref[0] + q_ref[1]) + ALPHA * l2_ref[...]
    x = x[:, :L]
    m = jnp.max(x, axis=1, keepdims=True)
    e = jnp.exp(x - m)
    lse = jnp.log(jnp.sum(e, axis=1, keepdims=True)) + m
    out_ref[...] = x - lse


def _lsm(q, l2):
    return pl.pallas_call(
        _lsm_body,
        grid=(N // _R,),
        in_specs=[
            pl.BlockSpec((NC, _R, LP), lambda i: (0, i, 0)),
            pl.BlockSpec((_R, LP), lambda i: (i, 0)),
        ],
        out_specs=pl.BlockSpec((_R, L), lambda i: (i, 0)),
        out_shape=jax.ShapeDtypeStruct((N, L), jnp.float32),
    )(q, l2)


def _pad2d(x, m, dtype):
    return jnp.concatenate([x, jnp.zeros((m - x.shape[0],), dtype)]).reshape(-1, 128)


def kernel(feature_indices, feature_values, edge_indices, edge_weights, W1, b1, W2, b2):
    f_src = _pad2d(feature_indices[1], FPAD, jnp.int32)
    f_dst = _pad2d(feature_indices[0], FPAD, jnp.int32)
    f_w = _pad2d(feature_values, FPAD, jnp.float32)
    e_src = _pad2d(edge_indices[1], EPAD, jnp.int32)
    e_dst = _pad2d(edge_indices[0], EPAD, jnp.int32)
    e_w = _pad2d(edge_weights, EPAD, jnp.float32)

    p = _fspmm(f_src, f_dst, f_w, W1)

    w2p = jnp.pad(W2, ((0, 0), (0, LP - L)))
    b2p = jnp.pad(b2, (0, LP - L)).reshape(1, LP)
    l2 = _dense(p, b1.reshape(1, H), w2p, b2p)

    q = jnp.broadcast_to(0.5 * l2, (NC, N, LP))
    for _ in range(ITERS):
        q = _pspmm(e_src, e_dst, e_w, l2, q)
    return _lsm(q, l2)
